# Initial kernel scaffold; baseline (speedup 1.0000x reference)
import jax
import jax.numpy as jnp
from jax.experimental import pallas as pl


def kernel(f_atoms, f_bonds, a2b, b2a, b2revb, a_scope, W_i, W_h, W_o, b_o):
    raise NotImplementedError("write your pallas kernel here")


# speedup vs baseline: 1.5762x; 1.5762x over previous
import functools

import jax
import jax.numpy as jnp
from jax import lax
from jax.experimental import pallas as pl
from jax.experimental.pallas import tpu as pltpu
from jax.experimental.pallas import tpu_sc as plsc

NC = 2
NS = 16
NW = NC * NS
LANES = 16

DEPTH = 6
H = 128
HG = H // LANES


def _sc_mesh():
    return plsc.VectorSubcoreMesh(core_axis_name="c", subcore_axis_name="s")


CA = 4
MAX_NB = 32
CHUNK_IDX = CA * MAX_NB


def _seg_sum_kernel(n_atoms):
    n_chunks = n_atoms // CA
    n_iters = (n_chunks + NW - 1) // NW

    @functools.partial(
        pl.kernel,
        out_type=jax.ShapeDtypeStruct((n_atoms, H), jnp.float32),
        mesh=_sc_mesh(),
        scratch_types=[
            pltpu.VMEM((CHUNK_IDX,), jnp.int32),
            pltpu.VMEM((CHUNK_IDX, H), jnp.float32),
            pltpu.VMEM((CA, H), jnp.float32),
            pltpu.SemaphoreType.DMA,
        ],
    )
    def k(m_hbm, a2b_hbm, out_hbm, idx_v, rows_v, out_v, sem):
        wid = lax.axis_index("s") * NC + lax.axis_index("c")

        @pl.loop(0, n_iters)
        def _(it):
            c = it * NW + wid

            @pl.when(c < n_chunks)
            def _():
                pltpu.sync_copy(a2b_hbm.at[pl.ds(c * CHUNK_IDX, CHUNK_IDX)],
                                idx_v)
                pltpu.async_copy(m_hbm.at[idx_v], rows_v, sem).wait()
                for a in range(CA):
                    def body(kk, accs, a=a):
                        row = a * MAX_NB + kk
                        return tuple(
                            accs[g] + rows_v[row, pl.ds(g * LANES, LANES)]
                            for g in range(HG))
                    accs = lax.fori_loop(
                        0, MAX_NB, body,
                        tuple(jnp.zeros((LANES,), jnp.float32)
                              for _ in range(HG)))
                    for g in range(HG):
                        out_v[a, pl.ds(g * LANES, LANES)] = accs[g]
                pltpu.sync_copy(out_v, out_hbm.at[pl.ds(c * CA, CA)])

    return k


CB = 128


def _gather_sub_kernel(n_bonds):
    n_chunks = n_bonds // CB
    n_iters = (n_chunks + NW - 1) // NW

    @functools.partial(
        pl.kernel,
        out_type=jax.ShapeDtypeStruct((n_bonds, H), jnp.float32),
        mesh=_sc_mesh(),
        scratch_types=[
            pltpu.VMEM((CB,), jnp.int32),
            pltpu.VMEM((CB,), jnp.int32),
            pltpu.VMEM((CB, H), jnp.float32),
            pltpu.VMEM((CB, H), jnp.float32),
            pltpu.SemaphoreType.DMA,
            pltpu.SemaphoreType.DMA,
        ],
    )
    def k(a_hbm, m_hbm, b2a_hbm, b2revb_hbm, out_hbm,
          idx1_v, idx2_v, ga_v, gm_v, sem1, sem2):
        wid = lax.axis_index("s") * NC + lax.axis_index("c")

        @pl.loop(0, n_iters)
        def _(it):
            c = it * NW + wid

            @pl.when(c < n_chunks)
            def _():
                base = c * CB
                pltpu.sync_copy(b2a_hbm.at[pl.ds(base, CB)], idx1_v)
                pltpu.sync_copy(b2revb_hbm.at[pl.ds(base, CB)], idx2_v)
                cp1 = pltpu.async_copy(a_hbm.at[idx1_v], ga_v, sem1)
                cp2 = pltpu.async_copy(m_hbm.at[idx2_v], gm_v, sem2)
                cp1.wait()
                cp2.wait()

                @pl.loop(0, CB)
                def _(r):
                    for g in range(HG):
                        sl = pl.ds(g * LANES, LANES)
                        ga_v[r, sl] = ga_v[r, sl] - gm_v[r, sl]

                pltpu.sync_copy(ga_v, out_hbm.at[pl.ds(base, CB)])

    return k


def _k1_call(f_bonds, w_i):
    n_bonds, fdim = f_bonds.shape
    br = 2560
    grid = (n_bonds // br,)

    def body(fb_ref, w_ref, inp_ref, m_ref):
        x = jnp.dot(fb_ref[...], w_ref[...],
                    preferred_element_type=jnp.float32)
        inp_ref[...] = x
        m_ref[...] = jnp.maximum(x, 0.0)

    return pl.pallas_call(
        body,
        grid=grid,
        in_specs=[
            pl.BlockSpec((br, fdim), lambda i: (i, 0)),
            pl.BlockSpec((fdim, H), lambda i: (0, 0)),
        ],
        out_specs=[
            pl.BlockSpec((br, H), lambda i: (i, 0)),
            pl.BlockSpec((br, H), lambda i: (i, 0)),
        ],
        out_shape=[
            jax.ShapeDtypeStruct((n_bonds, H), jnp.float32),
            jax.ShapeDtypeStruct((n_bonds, H), jnp.float32),
        ],
    )(f_bonds, w_i)


def _k3_call(t, inp, w_h):
    n_bonds = t.shape[0]
    br = 2560
    grid = (n_bonds // br,)

    def body(t_ref, i_ref, w_ref, m_ref):
        x = jnp.dot(t_ref[...], w_ref[...],
                    preferred_element_type=jnp.float32)
        m_ref[...] = jnp.maximum(i_ref[...] + x, 0.0)

    return pl.pallas_call(
        body,
        grid=grid,
        in_specs=[
            pl.BlockSpec((br, H), lambda i: (i, 0)),
            pl.BlockSpec((br, H), lambda i: (i, 0)),
            pl.BlockSpec((H, H), lambda i: (0, 0)),
        ],
        out_specs=pl.BlockSpec((br, H), lambda i: (i, 0)),
        out_shape=jax.ShapeDtypeStruct((n_bonds, H), jnp.float32),
    )(t, inp, w_h)


def _k4_call(f_atoms, a_msg, w_oa, w_om, b_o, n_mols, mol_size):
    n_atoms, fdim = f_atoms.shape
    mpb = 4
    apb = mpb * mol_size
    grid = (n_mols // mpb,)

    def body(fa_ref, am_ref, woa_ref, wom_ref, b_ref, out_ref):
        h = jnp.dot(fa_ref[...], woa_ref[...],
                    preferred_element_type=jnp.float32)
        h = h + jnp.dot(am_ref[...], wom_ref[...],
                        preferred_element_type=jnp.float32)
        h = jnp.maximum(h + b_ref[...], 0.0)
        inv = 1.0 / mol_size
        for m in range(mpb):
            s = jnp.sum(h[m * mol_size:(m + 1) * mol_size, :], axis=0) * inv
            out_ref[0, m, :] = s

    out = pl.pallas_call(
        body,
        grid=grid,
        in_specs=[
            pl.BlockSpec((apb, fdim), lambda i: (i, 0)),
            pl.BlockSpec((apb, H), lambda i: (i, 0)),
            pl.BlockSpec((fdim, H), lambda i: (0, 0)),
            pl.BlockSpec((H, H), lambda i: (0, 0)),
            pl.BlockSpec((1, H), lambda i: (0, 0)),
        ],
        out_specs=pl.BlockSpec((1, mpb, H), lambda i: (i, 0, 0)),
        out_shape=jax.ShapeDtypeStruct((n_mols // mpb, mpb, H), jnp.float32),
    )(f_atoms, a_msg, w_oa, w_om, b_o)
    return out.reshape(n_mols, H)


def kernel(f_atoms, f_bonds, a2b, b2a, b2revb, a_scope, W_i, W_h, W_o, b_o):
    n_atoms, fdim_a = f_atoms.shape
    n_bonds = f_bonds.shape[0]
    n_mols = a_scope.shape[0]
    mol_size = n_atoms // n_mols

    a2b_flat = a2b.reshape(-1)
    seg_sum = _seg_sum_kernel(n_atoms)
    gather_sub = _gather_sub_kernel(n_bonds)

    inp, msg = _k1_call(f_bonds, W_i)
    for _ in range(DEPTH - 1):
        a_msg = seg_sum(msg, a2b_flat)
        t = gather_sub(a_msg, msg, b2a, b2revb)
        msg = _k3_call(t, inp, W_h)

    a_msg = seg_sum(msg, a2b_flat)
    w_oa = W_o[:fdim_a]
    w_om = W_o[fdim_a:]
    return _k4_call(f_atoms, a_msg, w_oa, w_om, b_o.reshape(1, H),
                    n_mols, mol_size)
